# baseline (speedup 1.0000x reference)
import functools

import jax
import jax.numpy as jnp
from jax import lax
from jax.experimental import pallas as pl
from jax.experimental.pallas import tpu as pltpu
from jax.experimental.pallas import tpu_sc as plsc

_NUM_CORES = 2
_NUM_SUBCORES = 16
_NUM_WORKERS = _NUM_CORES * _NUM_SUBCORES
_LANES = 16


def kernel(data, indices):
    num_rows, row_dim = data.shape
    batch = indices.shape[0]
    idx = indices.reshape(batch).astype(jnp.int32)
    packed = data.reshape(num_rows // 2, 2 * row_dim)
    b_per_w = batch // _NUM_WORKERS

    mesh = plsc.VectorSubcoreMesh(core_axis_name="c", subcore_axis_name="s")

    @functools.partial(
        pl.kernel,
        mesh=mesh,
        out_type=jax.ShapeDtypeStruct((batch, row_dim), data.dtype),
        compiler_params=pltpu.CompilerParams(needs_layout_passes=False),
        scratch_types=[
            pltpu.VMEM((b_per_w,), jnp.int32),
            pltpu.VMEM((b_per_w,), jnp.int32),
            pltpu.VMEM((b_per_w // 2, 2 * row_dim), jnp.float32),
            pltpu.VMEM((b_per_w, row_dim), jnp.float32),
            pltpu.SemaphoreType.DMA,
        ],
    )
    def gather_rows_sc(packed_hbm, idx_hbm, out_hbm, idx_v, pair_v, rows_v,
                       out_v, sem):
        chunk = b_per_w // 2
        wid = lax.axis_index("s") * _NUM_CORES + lax.axis_index("c")
        base = wid * b_per_w
        pltpu.sync_copy(idx_hbm.at[pl.ds(base, b_per_w)], idx_v)

        @pl.loop(0, b_per_w, step=_LANES)
        def _(g):
            iv = idx_v[pl.ds(g, _LANES)]
            pair_v[pl.ds(g, _LANES)] = iv >> 1

        row_iota = lax.iota(jnp.int32, _LANES)
        zero_v = jnp.zeros((_LANES,), jnp.int32)

        for c in range(2):
            pltpu.async_copy(
                packed_hbm.at[pair_v.at[pl.ds(c * chunk, chunk)]], rows_v, sem
            ).wait()

            @pl.loop(0, chunk, step=_LANES)
            def _(g):
                iv = idx_v[pl.ds(c * chunk + g, _LANES)]
                col0 = (iv & 1) * row_dim
                rows = row_iota + g

                @pl.loop(0, row_dim, step=4)
                def _(j):
                    for u in range(4):
                        v = plsc.load_gather(rows_v, [rows, col0 + (j + u)])
                        plsc.store_scatter(
                            out_v,
                            [rows + (c * chunk), zero_v + (j + u)],
                            v,
                        )

        pltpu.sync_copy(out_v, out_hbm.at[pl.ds(base, b_per_w)])

    return gather_rows_sc(packed, idx)
